# baseline (speedup 1.0000x reference)
import functools

import jax
import jax.numpy as jnp
from jax import lax
from jax.experimental import pallas as pl
from jax.experimental.pallas import tpu as pltpu
from jax.experimental.pallas import tpu_sc as plsc

N_USERS = 100000
N_ITEMS = 100000
N_OCC = 64
MAX_TS = 1024
D = 32
B = 16384
L = 50

NC = 2
NS = 16
NW = NC * NS
LANES = 16

ROWS_W = B // NW
CHUNK = 32
N_CHUNKS = ROWS_W // CHUNK
ITEMS_CHUNK = CHUNK * L
GW = 40
NG = ITEMS_CHUNK // GW
IDX_ROWS = B * L // GW
IDX_ROWS_W = IDX_ROWS // NW


def _body(user_code, item_code_r, user_occ, ts_rank,
          user_emb, item_emb, occ_emb, user_temp_emb, temp_emb,
          user_bias, item_bias, bias,
          out_hbm,
          uc_all, oc_all, ts_all, bias_sv, rc_row,
          idx_c0, idx_c1, rows0, rows1, ibv0, ibv1, ue0, ue1, oe0, oe1,
          ute0, ute1, te0, te1, ubv0, ubv1, outv0, outv1,
          sem_in0, sem_in1, sem_out0, sem_out1):
  idx_c = [idx_c0, idx_c1]
  rows_2d = [rows0, rows1]
  ib_v = [ibv0, ibv1]
  ue_2d = [ue0, ue1]
  oe_2d = [oe0, oe1]
  ute_2d = [ute0, ute1]
  te_2d = [te0, te1]
  ub_v = [ubv0, ubv1]
  out_v = [outv0, outv1]
  sem_in = [sem_in0, sem_in1]
  sem_out = [sem_out0, sem_out1]
  wid = lax.axis_index("s") * NC + lax.axis_index("c")
  base_w = wid * ROWS_W

  pltpu.sync_copy(user_code.at[pl.ds(base_w, ROWS_W)], uc_all)
  pltpu.sync_copy(user_occ.at[pl.ds(base_w, ROWS_W)], oc_all)
  pltpu.sync_copy(ts_rank.at[pl.ds(base_w, ROWS_W)], ts_all)
  pltpu.sync_copy(bias, bias_sv.at[pl.ds(0, 1)])

  iota = lax.broadcasted_iota(jnp.int32, (LANES,), 0)
  lane15 = iota == (LANES - 1)
  lane0 = iota == 0
  zeros16 = jnp.zeros((LANES,), jnp.int32)
  bias_vec = plsc.load_gather(bias_sv, [zeros16])

  def in_copies(c, s, fire):
    f = pltpu.async_copy if fire else pltpu.make_async_copy
    cps = []
    for j in range(NG):
      cps.append(f(item_emb.at[idx_c[s].at[j]],
                   rows_2d[s].at[pl.ds(j * GW, GW)], sem_in[s]))
    for j in range(NG):
      cps.append(f(item_bias.at[idx_c[s].at[j]],
                   ib_v[s].at[pl.ds(j * GW, GW)], sem_in[s]))
    uc = uc_all.at[pl.ds(c * CHUNK, CHUNK)]
    cps.append(f(user_emb.at[uc], ue_2d[s], sem_in[s]))
    cps.append(f(user_temp_emb.at[uc], ute_2d[s], sem_in[s]))
    cps.append(f(user_bias.at[uc], ub_v[s], sem_in[s]))
    cps.append(f(occ_emb.at[oc_all.at[pl.ds(c * CHUNK, CHUNK)]],
                 oe_2d[s], sem_in[s]))
    cps.append(f(temp_emb.at[ts_all.at[pl.ds(c * CHUNK, CHUNK)]],
                 te_2d[s], sem_in[s]))
    return cps

  def fire(c, s):
    pltpu.sync_copy(item_code_r.at[pl.ds(wid * IDX_ROWS_W + c * NG, NG)],
                    idx_c[s])
    return in_copies(c, s, True)

  def drain_in(c, s):
    for cp in in_copies(c, s, False):
      cp.wait()

  def out_dma(c, s):
    return pltpu.make_async_copy(
        out_v[s],
        out_hbm.at[pl.ds((base_w + c * CHUNK) * L, CHUNK * L)], sem_out[s])

  perms = [jnp.bitwise_xor(iota, sh)[:, None] for sh in (1, 2, 4, 8)]
  _dnums = lax.GatherDimensionNumbers(
      offset_dims=(), collapsed_slice_dims=(0,), start_index_map=(0,))

  def allsum(p):
    for pm in perms:
      p = p + lax.gather(p, pm, _dnums, (1,),
                         mode=lax.GatherScatterMode.PROMISE_IN_BOUNDS)
    return p

  def compute(c, s):
    @pl.when(c >= 2)
    def _():
      out_dma(c - 2, s).wait()

    def row_body(r, _):
      q0 = ue_2d[s][r, pl.ds(0, LANES)] + oe_2d[s][r, pl.ds(0, LANES)]
      q1 = (ue_2d[s][r, pl.ds(LANES, LANES)] +
            oe_2d[s][r, pl.ds(LANES, LANES)])
      tp = (ute_2d[s][r, pl.ds(0, LANES)] * te_2d[s][r, pl.ds(0, LANES)] +
            ute_2d[s][r, pl.ds(LANES, LANES)] *
            te_2d[s][r, pl.ds(LANES, LANES)])
      rcv = (plsc.load_gather(ub_v[s], [jnp.full((LANES,), 0, jnp.int32) + r])
             + bias_vec)
      base_p = tp + jnp.where(lane0, rcv, jnp.zeros((LANES,), jnp.float32))

      def item_body(l, _):
        i = r * L + l
        i_vec = jnp.full((LANES,), 0, jnp.int32) + i
        ibs = plsc.load_gather(ib_v[s], [i_vec])
        p = (rows_2d[s][i, pl.ds(0, LANES)] * q0 +
             rows_2d[s][i, pl.ds(LANES, LANES)] * q1 +
             base_p +
             jnp.where(lane0, ibs, jnp.zeros((LANES,), jnp.float32)))
        plsc.store_scatter(out_v[s], [i_vec], allsum(p), mask=lane0)
        return 0

      lax.fori_loop(0, L, item_body, 0, unroll=8)
      return 0

    lax.fori_loop(0, CHUNK, row_body, 0, unroll=False)

    pltpu.async_copy(
        out_v[s],
        out_hbm.at[pl.ds((base_w + c * CHUNK) * L, CHUNK * L)], sem_out[s])

  fire(0, 0)

  def pair_body(c2, carry):
    c0 = 2 * c2
    fire(c0 + 1, 1)
    drain_in(c0, 0)
    compute(c0, 0)

    @pl.when(c2 < N_CHUNKS // 2 - 1)
    def _():
      fire(c0 + 2, 0)

    drain_in(c0 + 1, 1)
    compute(c0 + 1, 1)
    return carry

  lax.fori_loop(0, N_CHUNKS // 2, pair_body, 0, unroll=False)

  out_dma(N_CHUNKS - 2, 0).wait()
  out_dma(N_CHUNKS - 1, 1).wait()


@jax.jit
def _run(user_code, item_code_r, user_occ, ts_rank,
         user_emb, item_emb, occ_emb, user_temp_emb, temp_emb,
         user_bias, item_bias, bias):
  mesh = plsc.VectorSubcoreMesh(core_axis_name="c", subcore_axis_name="s",
                                num_cores=NC, num_subcores=NS)
  f = functools.partial(
      pl.kernel,
      out_type=jax.ShapeDtypeStruct((B * L,), jnp.float32),
      mesh=mesh,
      compiler_params=pltpu.CompilerParams(needs_layout_passes=False,
                                           use_tc_tiling_on_sc=False),
      scratch_types=[
          pltpu.VMEM((ROWS_W,), jnp.int32),
          pltpu.VMEM((ROWS_W,), jnp.int32),
          pltpu.VMEM((ROWS_W,), jnp.int32),
          pltpu.VMEM((LANES,), jnp.float32),
          pltpu.VMEM((CHUNK,), jnp.float32),
          pltpu.VMEM((NG, GW), jnp.int32),
          pltpu.VMEM((NG, GW), jnp.int32),
          pltpu.VMEM((ITEMS_CHUNK, D), jnp.float32),
          pltpu.VMEM((ITEMS_CHUNK, D), jnp.float32),
          pltpu.VMEM((ITEMS_CHUNK,), jnp.float32),
          pltpu.VMEM((ITEMS_CHUNK,), jnp.float32),
          pltpu.VMEM((CHUNK, D), jnp.float32),
          pltpu.VMEM((CHUNK, D), jnp.float32),
          pltpu.VMEM((CHUNK, D), jnp.float32),
          pltpu.VMEM((CHUNK, D), jnp.float32),
          pltpu.VMEM((CHUNK, D), jnp.float32),
          pltpu.VMEM((CHUNK, D), jnp.float32),
          pltpu.VMEM((CHUNK, D), jnp.float32),
          pltpu.VMEM((CHUNK, D), jnp.float32),
          pltpu.VMEM((CHUNK,), jnp.float32),
          pltpu.VMEM((CHUNK,), jnp.float32),
          pltpu.VMEM((ITEMS_CHUNK,), jnp.float32),
          pltpu.VMEM((ITEMS_CHUNK,), jnp.float32),
          pltpu.SemaphoreType.DMA,
          pltpu.SemaphoreType.DMA,
          pltpu.SemaphoreType.DMA,
          pltpu.SemaphoreType.DMA,
      ],
  )(_body)
  return f(user_code, item_code_r, user_occ, ts_rank,
           user_emb, item_emb, occ_emb, user_temp_emb, temp_emb,
           user_bias, item_bias, bias)


def kernel(user_code, item_code, user_occupation, item_timestamp_rank,
           user_emb, item_emb, occ_emb, user_temp_emb, temp_emb,
           user_bias, item_bias, bias):
  item_code_r = item_code.reshape(IDX_ROWS, GW)
  out = _run(user_code, item_code_r, user_occupation, item_timestamp_rank,
             user_emb, item_emb, occ_emb, user_temp_emb, temp_emb,
             user_bias, item_bias, bias)
  return out.reshape(B, L)


# speedup vs baseline: 2.1126x; 2.1126x over previous
import functools

import jax
import jax.numpy as jnp
from jax import lax
from jax.experimental import pallas as pl
from jax.experimental.pallas import tpu as pltpu
from jax.experimental.pallas import tpu_sc as plsc

N_USERS = 100000
N_ITEMS = 100000
N_OCC = 64
MAX_TS = 1024
D = 32
B = 16384
L = 50

NC = 2
NS = 16
NW = NC * NS
LANES = 16

ROWS_W = B // NW
CHUNK = 32
N_CHUNKS = ROWS_W // CHUNK
ITEMS_CHUNK = CHUNK * L
GW = 40
NG = ITEMS_CHUNK // GW
IDX_ROWS = B * L // GW
IDX_ROWS_W = IDX_ROWS // NW


def _body(user_code, item_code_r, user_occ, ts_rank,
          user_emb, item_emb, occ_emb, user_temp_emb, temp_emb,
          user_bias, item_bias, bias,
          out_hbm,
          uc_all, oc_all, ts_all, bias_sv, rc_row,
          idx_c0, idx_c1, rows0, rows1, ibv0, ibv1, ue0, ue1, oe0, oe1,
          ute0, ute1, te0, te1, ubv0, ubv1, outv0, outv1,
          sem_in0, sem_in1, sem_out0, sem_out1):
  idx_c = [idx_c0, idx_c1]
  rows_2d = [rows0, rows1]
  ib_v = [ibv0, ibv1]
  ue_2d = [ue0, ue1]
  oe_2d = [oe0, oe1]
  ute_2d = [ute0, ute1]
  te_2d = [te0, te1]
  ub_v = [ubv0, ubv1]
  out_v = [outv0, outv1]
  sem_in = [sem_in0, sem_in1]
  sem_out = [sem_out0, sem_out1]
  wid = lax.axis_index("s") * NC + lax.axis_index("c")
  base_w = wid * ROWS_W

  pltpu.sync_copy(user_code.at[pl.ds(base_w, ROWS_W)], uc_all)
  pltpu.sync_copy(user_occ.at[pl.ds(base_w, ROWS_W)], oc_all)
  pltpu.sync_copy(ts_rank.at[pl.ds(base_w, ROWS_W)], ts_all)
  pltpu.sync_copy(bias, bias_sv.at[pl.ds(0, 1)])

  iota = lax.broadcasted_iota(jnp.int32, (LANES,), 0)
  lane15 = iota == (LANES - 1)
  lane0 = iota == 0
  zeros16 = jnp.zeros((LANES,), jnp.int32)
  bias_vec = plsc.load_gather(bias_sv, [zeros16])

  def in_copies(c, s, fire):
    f = pltpu.async_copy if fire else pltpu.make_async_copy
    cps = []
    for j in range(NG):
      cps.append(f(item_emb.at[idx_c[s].at[j]],
                   rows_2d[s].at[pl.ds(j * GW, GW)], sem_in[s]))
    for j in range(NG):
      cps.append(f(item_bias.at[idx_c[s].at[j]],
                   ib_v[s].at[pl.ds(j * GW, GW)], sem_in[s]))
    uc = uc_all.at[pl.ds(c * CHUNK, CHUNK)]
    cps.append(f(user_emb.at[uc], ue_2d[s], sem_in[s]))
    cps.append(f(user_temp_emb.at[uc], ute_2d[s], sem_in[s]))
    cps.append(f(user_bias.at[uc], ub_v[s], sem_in[s]))
    cps.append(f(occ_emb.at[oc_all.at[pl.ds(c * CHUNK, CHUNK)]],
                 oe_2d[s], sem_in[s]))
    cps.append(f(temp_emb.at[ts_all.at[pl.ds(c * CHUNK, CHUNK)]],
                 te_2d[s], sem_in[s]))
    return cps

  def fire(c, s):
    pltpu.sync_copy(item_code_r.at[pl.ds(wid * IDX_ROWS_W + c * NG, NG)],
                    idx_c[s])
    return in_copies(c, s, True)

  def drain_in(c, s):
    for cp in in_copies(c, s, False):
      cp.wait()

  def out_dma(c, s):
    return pltpu.make_async_copy(
        out_v[s].at[pl.ds(0, ITEMS_CHUNK)],
        out_hbm.at[pl.ds((base_w + c * CHUNK) * L, CHUNK * L)], sem_out[s])

  perms = [jnp.bitwise_xor(iota, sh)[:, None] for sh in (1, 2, 4, 8)]
  _dnums = lax.GatherDimensionNumbers(
      offset_dims=(), collapsed_slice_dims=(0,), start_index_map=(0,))

  def allsum(p):
    for pm in perms:
      p = p + lax.gather(p, pm, _dnums, (1,),
                         mode=lax.GatherScatterMode.PROMISE_IN_BOUNDS)
    return p

  lane_is = [iota == j for j in range(LANES)]
  zf = jnp.zeros((LANES,), jnp.float32)

  def compute(c, s):
    @pl.when(c >= 2)
    def _():
      out_dma(c - 2, s).wait()

    def row_body(r, _):
      q0 = ue_2d[s][r, pl.ds(0, LANES)] + oe_2d[s][r, pl.ds(0, LANES)]
      q1 = (ue_2d[s][r, pl.ds(LANES, LANES)] +
            oe_2d[s][r, pl.ds(LANES, LANES)])
      tp = (ute_2d[s][r, pl.ds(0, LANES)] * te_2d[s][r, pl.ds(0, LANES)] +
            ute_2d[s][r, pl.ds(LANES, LANES)] *
            te_2d[s][r, pl.ds(LANES, LANES)])
      base_all = (plsc.load_gather(ub_v[s],
                                   [jnp.full((LANES,), 0, jnp.int32) + r])
                  + bias_vec + allsum(tp))

      for k in range(0, L, LANES):
        nj = min(LANES, L - k)
        res = zf
        for j in range(nj):
          i = r * L + k + j
          p = (rows_2d[s][i, pl.ds(0, LANES)] * q0 +
               rows_2d[s][i, pl.ds(LANES, LANES)] * q1)
          res = jnp.where(lane_is[j], allsum(p), res)
        i_vec = (jnp.full((LANES,), 0, jnp.int32) + (r * L + k)) + iota
        res = res + base_all + plsc.load_gather(ib_v[s], [i_vec])
        plsc.store_scatter(out_v[s], [i_vec], res)
      return 0

    lax.fori_loop(0, CHUNK, row_body, 0, unroll=False)

    pltpu.async_copy(
        out_v[s].at[pl.ds(0, ITEMS_CHUNK)],
        out_hbm.at[pl.ds((base_w + c * CHUNK) * L, CHUNK * L)], sem_out[s])

  fire(0, 0)

  def pair_body(c2, carry):
    c0 = 2 * c2
    fire(c0 + 1, 1)
    drain_in(c0, 0)
    compute(c0, 0)

    @pl.when(c2 < N_CHUNKS // 2 - 1)
    def _():
      fire(c0 + 2, 0)

    drain_in(c0 + 1, 1)
    compute(c0 + 1, 1)
    return carry

  lax.fori_loop(0, N_CHUNKS // 2, pair_body, 0, unroll=False)

  out_dma(N_CHUNKS - 2, 0).wait()
  out_dma(N_CHUNKS - 1, 1).wait()


@jax.jit
def _run(user_code, item_code_r, user_occ, ts_rank,
         user_emb, item_emb, occ_emb, user_temp_emb, temp_emb,
         user_bias, item_bias, bias):
  mesh = plsc.VectorSubcoreMesh(core_axis_name="c", subcore_axis_name="s",
                                num_cores=NC, num_subcores=NS)
  f = functools.partial(
      pl.kernel,
      out_type=jax.ShapeDtypeStruct((B * L,), jnp.float32),
      mesh=mesh,
      compiler_params=pltpu.CompilerParams(needs_layout_passes=False,
                                           use_tc_tiling_on_sc=False),
      scratch_types=[
          pltpu.VMEM((ROWS_W,), jnp.int32),
          pltpu.VMEM((ROWS_W,), jnp.int32),
          pltpu.VMEM((ROWS_W,), jnp.int32),
          pltpu.VMEM((LANES,), jnp.float32),
          pltpu.VMEM((CHUNK,), jnp.float32),
          pltpu.VMEM((NG, GW), jnp.int32),
          pltpu.VMEM((NG, GW), jnp.int32),
          pltpu.VMEM((ITEMS_CHUNK, D), jnp.float32),
          pltpu.VMEM((ITEMS_CHUNK, D), jnp.float32),
          pltpu.VMEM((ITEMS_CHUNK + LANES,), jnp.float32),
          pltpu.VMEM((ITEMS_CHUNK + LANES,), jnp.float32),
          pltpu.VMEM((CHUNK, D), jnp.float32),
          pltpu.VMEM((CHUNK, D), jnp.float32),
          pltpu.VMEM((CHUNK, D), jnp.float32),
          pltpu.VMEM((CHUNK, D), jnp.float32),
          pltpu.VMEM((CHUNK, D), jnp.float32),
          pltpu.VMEM((CHUNK, D), jnp.float32),
          pltpu.VMEM((CHUNK, D), jnp.float32),
          pltpu.VMEM((CHUNK, D), jnp.float32),
          pltpu.VMEM((CHUNK,), jnp.float32),
          pltpu.VMEM((CHUNK,), jnp.float32),
          pltpu.VMEM((ITEMS_CHUNK + LANES,), jnp.float32),
          pltpu.VMEM((ITEMS_CHUNK + LANES,), jnp.float32),
          pltpu.SemaphoreType.DMA,
          pltpu.SemaphoreType.DMA,
          pltpu.SemaphoreType.DMA,
          pltpu.SemaphoreType.DMA,
      ],
  )(_body)
  return f(user_code, item_code_r, user_occ, ts_rank,
           user_emb, item_emb, occ_emb, user_temp_emb, temp_emb,
           user_bias, item_bias, bias)


def kernel(user_code, item_code, user_occupation, item_timestamp_rank,
           user_emb, item_emb, occ_emb, user_temp_emb, temp_emb,
           user_bias, item_bias, bias):
  item_code_r = item_code.reshape(IDX_ROWS, GW)
  out = _run(user_code, item_code_r, user_occupation, item_timestamp_rank,
             user_emb, item_emb, occ_emb, user_temp_emb, temp_emb,
             user_bias, item_bias, bias)
  return out.reshape(B, L)
